# baseline (speedup 1.0000x reference)
import functools

import jax
import jax.numpy as jnp
from jax.experimental import pallas as pl

P = 512
H = 64
E = 32
N = P * P


def _lstm_block_kernel(x_ref, y_ref, nei_ref, ht_ref, ct_ref,
                       w0_ref, w1_ref, bemb_ref, wih_ref, whh_ref, bias_ref,
                       ho_ref, co_ref):
    x = x_ref[...]
    y = y_ref[...]
    ht = ht_ref[...]
    ct = ct_ref[...]
    w0 = w0_ref[...]
    w1 = w1_ref[...]
    bemb = bemb_ref[...]
    emb = jnp.maximum(x * w0 + y * w1 + bemb, 0.0)
    gates = (jnp.dot(emb, wih_ref[...], preferred_element_type=jnp.float32)
             + jnp.dot(ht, whh_ref[...], preferred_element_type=jnp.float32)
             + bias_ref[...])
    i = jax.nn.sigmoid(gates[:, 0 * H:1 * H])
    f = jax.nn.sigmoid(gates[:, 1 * H:2 * H])
    g = jnp.tanh(gates[:, 2 * H:3 * H])
    o = jax.nn.sigmoid(gates[:, 3 * H:4 * H])
    c_new = f * ct + i * g
    h_new = o * jnp.tanh(c_new)
    m = nei_ref[...] > 0
    ho_ref[...] = jnp.where(m, h_new, ht)
    co_ref[...] = jnp.where(m, c_new, ct)


@functools.partial(jax.jit, static_argnames=("rows",))
def _run(corr_index, rela_ht, rela_ct, nei_index,
         W_emb, b_emb, W_ih, W_hh, b_ih, b_hh, rows=1024):
    x = corr_index[:, :, 0].reshape(N, 1)
    y = corr_index[:, :, 1].reshape(N, 1)
    ht = rela_ht.reshape(N, H)
    ct = rela_ct.reshape(N, H)
    nei = nei_index.reshape(N, 1)
    w0 = W_emb[:, 0].reshape(1, E)
    w1 = W_emb[:, 1].reshape(1, E)
    bemb = b_emb.reshape(1, E)
    wih = W_ih.T
    whh = W_hh.T
    bias = (b_ih + b_hh).reshape(1, 4 * H)

    grid = (N // rows,)
    row_spec = lambda c: pl.BlockSpec((rows, c), lambda i: (i, 0))
    full_spec = lambda r, c: pl.BlockSpec((r, c), lambda i: (0, 0))
    ho, co = pl.pallas_call(
        _lstm_block_kernel,
        grid=grid,
        in_specs=[
            row_spec(1),
            row_spec(1),
            row_spec(1),
            row_spec(H),
            row_spec(H),
            full_spec(1, E),
            full_spec(1, E),
            full_spec(1, E),
            full_spec(E, 4 * H),
            full_spec(H, 4 * H),
            full_spec(1, 4 * H),
        ],
        out_specs=[row_spec(H), row_spec(H)],
        out_shape=[
            jax.ShapeDtypeStruct((N, H), jnp.float32),
            jax.ShapeDtypeStruct((N, H), jnp.float32),
        ],
    )(x, y, nei, ht, ct, w0, w1, bemb, wih, whh, bias)
    return ho.reshape(P, P, H), co.reshape(P, P, H)


def kernel(corr_index, rela_ht, rela_ct, nei_index,
           W_emb, b_emb, W_ih, W_hh, b_ih, b_hh):
    return _run(corr_index, rela_ht, rela_ct, nei_index,
                W_emb, b_emb, W_ih, W_hh, b_ih, b_hh)
